# baseline (speedup 1.0000x reference)
import functools

import jax
import jax.numpy as jnp
from jax import lax
from jax.experimental import pallas as pl
from jax.experimental.pallas import tpu as pltpu
from jax.experimental.pallas import tpu_sc as plsc

B = 16
T = 4096
D = 64
L = 128
V = 1000000
NC = 2
NS = 16
RPC = B // NC
HALF = T // 2
CH = 128
NCHUNK = HALF // CH
NBUF = 4
SLOT = 2 * (L + 1) + 6


def _body(para, slen, zeros, w2, out, len_v, bnd_v, mark_v, seg2d, tok_v, tok2_v,
          rows_b, cmb_v, out_v, acc_sh, sem_g, sem_s):
    c = lax.axis_index("c")
    s = lax.axis_index("s")
    slot = lax.rem(s, RPC)
    h = s // RPC
    b = c * RPC + slot
    t0 = h * HALF

    @pl.when(h == 0)
    def _():
        pltpu.sync_copy(zeros, acc_sh.at[pl.ds(slot * SLOT, SLOT)])

    pltpu.sync_copy(slen.at[b], len_v)
    pltpu.sync_copy(para.at[b, pl.ds(t0, HALF)], tok_v)

    carry = jnp.int32(0)
    for k in range(L // 16):
        v = len_v[pl.ds(k * 16, 16)]
        bnd_v[pl.ds(k * 16, 16)] = plsc.cumsum(v) + carry
        carry = carry + jnp.sum(v)

    off = jnp.int32(0)
    for k in range(L // 16):
        bv = bnd_v[pl.ds(k * 16, 16)]
        off = off + jnp.sum((bv <= t0 - 1).astype(jnp.int32))

    zero16 = jnp.zeros((16,), jnp.int32)
    for k in range(HALF // 16):
        mark_v[pl.ds(k * 16, 16)] = zero16

    lane = lax.iota(jnp.int32, 16)
    shift_idx = jnp.minimum(lane + 1, 15)
    for k in range(L // 16):
        v = bnd_v[pl.ds(k * 16, 16)]
        nxt = v.at[shift_idx].get(mode="promise_in_bounds")
        if k < L // 16 - 1:
            nv = bnd_v[pl.ds((k + 1) * 16, 16)]
            nxt = jnp.where(lane == 15, nv[0], nxt)
        else:
            nxt = jnp.where(lane == 15, jnp.int32(0x7FFFFFFF), nxt)
        rel = v - t0
        m = (v != nxt) & (rel >= 0) & (rel < HALF)
        relc = jnp.clip(rel, 0, HALF - 1)
        plsc.store_scatter(mark_v, [relc], lane + (16 * k + 1), mask=m)

    carry2 = off
    base = slot * SLOT
    for k in range(HALF // 16):
        v = mark_v[pl.ds(k * 16, 16)]
        cm = jnp.maximum(plsc.cummax(v), carry2)
        tv = tok_v[pl.ds(k * 16, 16)]
        tok2_v[pl.ds(k * 16, 16)] = jnp.right_shift(tv, 1)
        seg2d[k // 8, pl.ds((k % 8) * 16, 16)] = (
            cm * 2 + base + jnp.bitwise_and(tv, 1)
        )
        carry2 = jnp.max(cm)

    plsc.subcore_barrier()

    def gstart(ch):
        idx = tok2_v.at[pl.ds(ch * CH, CH)]
        return pltpu.async_copy(w2.at[idx], rows_b.at[ch % NBUF], sem_g)

    gd = [None] * NCHUNK
    sd = [None] * NCHUNK
    for i in range(min(NBUF - 2, NCHUNK)):
        gd[i] = gstart(i)
    for ch in range(NCHUNK):
        if ch >= 2:
            sd[ch - 2].wait()
        nxt = ch + NBUF - 2
        if nxt < NCHUNK:
            gd[nxt] = gstart(nxt)
        gd[ch].wait()
        sd[ch] = pltpu.async_copy(
            rows_b.at[ch % NBUF], acc_sh.at[seg2d.at[ch]], sem_s, add=True
        )
    for ch in range(max(NCHUNK - 2, 0), NCHUNK):
        sd[ch].wait()

    plsc.subcore_barrier()

    pltpu.sync_copy(acc_sh.at[pl.ds(slot * SLOT + h * L, L)], cmb_v)
    for ll in range(L // 2):
        for j in range(D // 16):
            out_v[ll, pl.ds(j * 16, 16)] = (
                cmb_v[2 * ll, pl.ds(j * 16, 16)]
                + cmb_v[2 * ll + 1, pl.ds(D + j * 16, 16)]
            )
    pltpu.sync_copy(out_v, out.at[b, pl.ds(h * (L // 2), L // 2)])


NFULL = V // 128
NTAIL = V - NFULL * 128
TBW = 32


def _tbody(wt, w2out, blk2, tb2, blk64, tb32, sem_i, sem_o):
    c = lax.axis_index("c")
    s = lax.axis_index("s")
    w = c * NS + s
    npair = (NFULL // TBW) // 2
    nextra = NFULL - TBW * 2 * npair

    lane = lax.iota(jnp.int32, 16)
    rows_k = [(jnp.int32(16 * k) + lane) >> 1 for k in range(8)]
    parcol_k = [jnp.bitwise_and(jnp.int32(16 * k) + lane, 1) * D for k in range(8)]

    def transpose_block(src, dst):
        @plsc.parallel_loop(0, D, unroll=8)
        def _(d):
            for k in range(8):
                vals = src[d, pl.ds(16 * k, 16)]
                plsc.store_scatter(dst, [rows_k[k], parcol_k[k] + d], vals)

    def drain_in():
        pltpu.make_async_copy(wt.at[:, pl.ds(0, 128)], blk2.at[0], sem_i).wait()

    def drain_out():
        pltpu.make_async_copy(wt.at[:, pl.ds(0, 128)], blk2.at[0], sem_o).wait()

    def issue_in(i, buf):
        bb = w + TBW * i
        pltpu.async_copy(wt.at[:, pl.ds(bb * 128, 128)], blk2.at[buf], sem_i)

    issue_in(jnp.int32(0), 0)
    issue_in(jnp.int32(1), 1)

    def body(p, _):
        i = 2 * p
        for q in range(2):
            bb = w + TBW * (i + q)

            @pl.when(i + q >= 2)
            def _():
                drain_out()

            drain_in()
            transpose_block(blk2.at[q], tb2.at[q])
            pltpu.async_copy(
                tb2.at[q].at[:, pl.ds(0, 2 * D)],
                w2out.at[pl.ds(bb * 64, 64)], sem_o,
            )

            @pl.when(i + q < 2 * npair - 2)
            def _():
                issue_in(i + q + 2, q)

        return 0

    lax.fori_loop(0, npair, body, 0)
    drain_out()
    drain_out()

    @pl.when(w < nextra)
    def _():
        bb = NFULL - nextra + w
        pltpu.sync_copy(wt.at[:, pl.ds(bb * 128, 128)], blk2.at[0])
        transpose_block(blk2.at[0], tb2.at[0])
        pltpu.sync_copy(tb2.at[0].at[:, pl.ds(0, 2 * D)],
                        w2out.at[pl.ds(bb * 64, 64)])

    @pl.when(w == TBW - 1)
    def _():
        pltpu.sync_copy(wt.at[:, pl.ds(NFULL * 128, NTAIL)], blk64)
        for d in range(D):
            for k in range(NTAIL // 16):
                vals = blk64[d, pl.ds(16 * k, 16)]
                plsc.store_scatter(tb32, [rows_k[k], parcol_k[k] + d], vals)
        pltpu.sync_copy(tb32.at[:, pl.ds(0, 2 * D)],
                        w2out.at[pl.ds(NFULL * 64, NTAIL // 2)])


@jax.jit
def _run(para, slen, wt):
    mesh = plsc.VectorSubcoreMesh(
        core_axis_name="c", subcore_axis_name="s", num_cores=NC, num_subcores=NS
    )
    trans = pl.kernel(
        _tbody,
        out_type=jax.ShapeDtypeStruct((V // 2, 2 * D), jnp.float32),
        mesh=mesh,
        compiler_params=pltpu.CompilerParams(needs_layout_passes=False),
        scratch_types=[
            pltpu.VMEM((2, D, 128), jnp.float32),
            pltpu.VMEM((2, 64, 2 * D + 1), jnp.float32),
            pltpu.VMEM((D, NTAIL), jnp.float32),
            pltpu.VMEM((NTAIL // 2, 2 * D + 1), jnp.float32),
            pltpu.SemaphoreType.DMA,
            pltpu.SemaphoreType.DMA,
        ],
    )
    w2 = trans(wt)
    mesh = plsc.VectorSubcoreMesh(
        core_axis_name="c", subcore_axis_name="s", num_cores=NC, num_subcores=NS
    )
    zeros = jnp.zeros((SLOT, 2 * D), jnp.float32)
    f = pl.kernel(
        _body,
        out_type=jax.ShapeDtypeStruct((B, L, D), jnp.float32),
        mesh=mesh,
        compiler_params=pltpu.CompilerParams(needs_layout_passes=False),
        scratch_types=[
            pltpu.VMEM((L,), jnp.int32),
            pltpu.VMEM((L,), jnp.int32),
            pltpu.VMEM((HALF,), jnp.int32),
            pltpu.VMEM((NCHUNK, CH), jnp.int32),
            pltpu.VMEM((HALF,), jnp.int32),
            pltpu.VMEM((HALF,), jnp.int32),
            pltpu.VMEM((NBUF, CH, 2 * D), jnp.float32),
            pltpu.VMEM((L, 2 * D), jnp.float32),
            pltpu.VMEM((L // 2, D), jnp.float32),
            pltpu.VMEM_SHARED((RPC * SLOT, 2 * D), jnp.float32),
            pltpu.SemaphoreType.DMA,
            pltpu.SemaphoreType.DMA,
        ],
    )
    return f(para, slen, zeros, w2)


def kernel(paragraph_variable, sentence_length_list, max_no_lines, W):
    del max_no_lines
    para = paragraph_variable.astype(jnp.int32)
    slen = sentence_length_list.astype(jnp.int32)
    return _run(para, slen, W.T)


# speedup vs baseline: 1.4354x; 1.4354x over previous
import functools

import jax
import jax.numpy as jnp
from jax import lax
from jax.experimental import pallas as pl
from jax.experimental.pallas import tpu as pltpu
from jax.experimental.pallas import tpu_sc as plsc

B = 16
T = 4096
D = 64
L = 128
NC = 2
NS = 16
RPC = B // NC
HALF = T // 2
CH = 128
NCHUNK = HALF // CH
NBUF = 8
ROWSTRIDE = L + 8


def _body(para, slen, zeros, w, out, len_v, bnd_v, mark_v, seg2d, tok_v, rows_b,
          acc_sh, sem_g, sem_s):
    c = lax.axis_index("c")
    s = lax.axis_index("s")
    slot = lax.rem(s, RPC)
    h = s // RPC
    b = c * RPC + slot
    t0 = h * HALF

    @pl.when(h == 0)
    def _():
        pltpu.sync_copy(zeros, acc_sh.at[pl.ds(slot * ROWSTRIDE, ROWSTRIDE)])

    pltpu.sync_copy(slen.at[b], len_v)
    pltpu.sync_copy(para.at[b, pl.ds(t0, HALF)], tok_v)

    carry = jnp.int32(0)
    for k in range(L // 16):
        v = len_v[pl.ds(k * 16, 16)]
        bnd_v[pl.ds(k * 16, 16)] = plsc.cumsum(v) + carry
        carry = carry + jnp.sum(v)

    off = jnp.int32(0)
    for k in range(L // 16):
        bv = bnd_v[pl.ds(k * 16, 16)]
        off = off + jnp.sum((bv <= t0 - 1).astype(jnp.int32))

    zero16 = jnp.zeros((16,), jnp.int32)
    for k in range(HALF // 16):
        mark_v[pl.ds(k * 16, 16)] = zero16

    lane = lax.iota(jnp.int32, 16)
    shift_idx = jnp.minimum(lane + 1, 15)
    for k in range(L // 16):
        v = bnd_v[pl.ds(k * 16, 16)]
        nxt = v.at[shift_idx].get(mode="promise_in_bounds")
        if k < L // 16 - 1:
            nv = bnd_v[pl.ds((k + 1) * 16, 16)]
            nxt = jnp.where(lane == 15, nv[0], nxt)
        else:
            nxt = jnp.where(lane == 15, jnp.int32(0x7FFFFFFF), nxt)
        rel = v - t0
        m = (v != nxt) & (rel >= 0) & (rel < HALF)
        relc = jnp.clip(rel, 0, HALF - 1)
        plsc.store_scatter(mark_v, [relc], lane + (16 * k + 1), mask=m)

    carry2 = off
    base = slot * ROWSTRIDE
    for k in range(HALF // 16):
        v = mark_v[pl.ds(k * 16, 16)]
        cm = jnp.maximum(plsc.cummax(v), carry2)
        seg2d[k // 8, pl.ds((k % 8) * 16, 16)] = cm + base
        carry2 = jnp.max(cm)

    plsc.subcore_barrier()

    def gstart(ch):
        idx = tok_v.at[pl.ds(ch * CH, CH)]
        return pltpu.async_copy(w.at[idx], rows_b.at[ch % NBUF], sem_g)

    gd = [None] * NCHUNK
    sd = [None] * NCHUNK
    for i in range(min(NBUF - 2, NCHUNK)):
        gd[i] = gstart(i)
    for ch in range(NCHUNK):
        if ch >= 2:
            sd[ch - 2].wait()
        nxt = ch + NBUF - 2
        if nxt < NCHUNK:
            gd[nxt] = gstart(nxt)
        gd[ch].wait()
        sd[ch] = pltpu.async_copy(
            rows_b.at[ch % NBUF], acc_sh.at[seg2d.at[ch]], sem_s, add=True
        )
    for ch in range(max(NCHUNK - 2, 0), NCHUNK):
        sd[ch].wait()

    plsc.subcore_barrier()

    @pl.when(h == 0)
    def _():
        pltpu.sync_copy(acc_sh.at[pl.ds(slot * ROWSTRIDE, L)], out.at[b])


@jax.jit
def _run(para, slen, w):
    mesh = plsc.VectorSubcoreMesh(
        core_axis_name="c", subcore_axis_name="s", num_cores=NC, num_subcores=NS
    )
    zeros = jnp.zeros((ROWSTRIDE, D), jnp.float32)
    f = pl.kernel(
        _body,
        out_type=jax.ShapeDtypeStruct((B, L, D), jnp.float32),
        mesh=mesh,
        compiler_params=pltpu.CompilerParams(needs_layout_passes=False, use_tc_tiling_on_sc=False),
        scratch_types=[
            pltpu.VMEM((L,), jnp.int32),
            pltpu.VMEM((L,), jnp.int32),
            pltpu.VMEM((HALF,), jnp.int32),
            pltpu.VMEM((NCHUNK, CH), jnp.int32),
            pltpu.VMEM((HALF,), jnp.int32),
            pltpu.VMEM((NBUF, CH, D), jnp.float32),
            pltpu.VMEM_SHARED((RPC * ROWSTRIDE, D), jnp.float32),
            pltpu.SemaphoreType.DMA,
            pltpu.SemaphoreType.DMA,
        ],
    )
    return f(para, slen, zeros, w)


def kernel(paragraph_variable, sentence_length_list, max_no_lines, W):
    del max_no_lines
    para = paragraph_variable.astype(jnp.int32)
    slen = sentence_length_list.astype(jnp.int32)
    return _run(para, slen, W)


# speedup vs baseline: 2.4547x; 1.7101x over previous
import functools

import jax
import jax.numpy as jnp
from jax import lax
from jax.experimental import pallas as pl
from jax.experimental.pallas import tpu as pltpu
from jax.experimental.pallas import tpu_sc as plsc

B = 16
T = 4096
D = 64
L = 128
V = 1000000
NC = 2
NS = 16
RPC = B // NC
HALF = T // 2
CH = 128
NCHUNK = HALF // CH
NBUF = 4
SLOT = 2 * (L + 1) + 6
VMAIN = (V // 4096) * 4096
UMAIN = VMAIN // 2
HT = (V - VMAIN) // 2


def _mbody(xa_ref, xb_ref, y_ref):
    y_ref[...] = jnp.concatenate(
        [jnp.transpose(xa_ref[...]), jnp.transpose(xb_ref[...])], axis=1)


def _tailbody(w2_ref, xt_ref, y_ref):
    del w2_ref
    xt = jnp.transpose(xt_ref[...])
    y_ref[0:HT, :] = jnp.concatenate([xt[0:HT], xt[HT:2 * HT]], axis=1)


def _build_w2(wt):
    w2a = pl.pallas_call(
        _mbody,
        grid=(VMAIN // 4096,),
        in_specs=[
            pl.BlockSpec((D, 2048), lambda i: (0, 2 * i)),
            pl.BlockSpec((D, 2048), lambda i: (0, 2 * i + 1)),
        ],
        out_specs=pl.BlockSpec((2048, 2 * D), lambda i: (i, 0)),
        out_shape=jax.ShapeDtypeStruct((V // 2, 2 * D), jnp.float32),
        compiler_params=pltpu.CompilerParams(
            dimension_semantics=("arbitrary",)),
    )(wt, wt)
    return pl.pallas_call(
        _tailbody,
        grid=(1,),
        in_specs=[
            pl.BlockSpec(memory_space=pltpu.HBM),
            pl.BlockSpec((D, 1024), lambda i: (0, VMAIN // 1024)),
        ],
        out_specs=pl.BlockSpec((512, 2 * D), lambda i: (UMAIN // 512, 0)),
        out_shape=jax.ShapeDtypeStruct((V // 2, 2 * D), jnp.float32),
        input_output_aliases={0: 0},
        compiler_params=pltpu.CompilerParams(
            dimension_semantics=("arbitrary",)),
    )(w2a, wt)


def _body(para, slen, zeros, w2, out, len_v, bnd_v, mark_v, seg2d, tok_v, tok2_v,
          rows_b, cmb_v, out_v, acc_sh, sem_g, sem_s):
    c = lax.axis_index("c")
    s = lax.axis_index("s")
    slot = lax.rem(s, RPC)
    h = s // RPC
    b = c * RPC + slot
    t0 = h * HALF

    @pl.when(h == 0)
    def _():
        pltpu.sync_copy(zeros, acc_sh.at[pl.ds(slot * SLOT, SLOT)])

    pltpu.sync_copy(slen.at[b], len_v)
    pltpu.sync_copy(para.at[b, pl.ds(t0, HALF)], tok_v)

    carry = jnp.int32(0)
    for k in range(L // 16):
        v = len_v[pl.ds(k * 16, 16)]
        bnd_v[pl.ds(k * 16, 16)] = plsc.cumsum(v) + carry
        carry = carry + jnp.sum(v)

    off = jnp.int32(0)
    for k in range(L // 16):
        bv = bnd_v[pl.ds(k * 16, 16)]
        off = off + jnp.sum((bv <= t0 - 1).astype(jnp.int32))

    zero16 = jnp.zeros((16,), jnp.int32)
    for k in range(HALF // 16):
        mark_v[pl.ds(k * 16, 16)] = zero16

    lane = lax.iota(jnp.int32, 16)
    shift_idx = jnp.minimum(lane + 1, 15)
    for k in range(L // 16):
        v = bnd_v[pl.ds(k * 16, 16)]
        nxt = v.at[shift_idx].get(mode="promise_in_bounds")
        if k < L // 16 - 1:
            nv = bnd_v[pl.ds((k + 1) * 16, 16)]
            nxt = jnp.where(lane == 15, nv[0], nxt)
        else:
            nxt = jnp.where(lane == 15, jnp.int32(0x7FFFFFFF), nxt)
        rel = v - t0
        m = (v != nxt) & (rel >= 0) & (rel < HALF)
        relc = jnp.clip(rel, 0, HALF - 1)
        plsc.store_scatter(mark_v, [relc], lane + (16 * k + 1), mask=m)

    carry2 = off
    base = slot * SLOT
    for k in range(HALF // 16):
        v = mark_v[pl.ds(k * 16, 16)]
        cm = jnp.maximum(plsc.cummax(v), carry2)
        tv = tok_v[pl.ds(k * 16, 16)]
        vm = tv < VMAIN
        um = ((tv >> 12) << 11) + jnp.bitwise_and(tv, 2047)
        r = tv - VMAIN
        pt = (r >= HT).astype(jnp.int32)
        ut = UMAIN + r - HT * pt
        tok2_v[pl.ds(k * 16, 16)] = jnp.where(vm, um, ut)
        p = jnp.where(vm, jnp.bitwise_and(tv >> 11, 1), pt)
        seg2d[k // 8, pl.ds((k % 8) * 16, 16)] = cm * 2 + base + p
        carry2 = jnp.max(cm)

    plsc.subcore_barrier()

    def gstart(ch):
        idx = tok2_v.at[pl.ds(ch * CH, CH)]
        return pltpu.async_copy(w2.at[idx], rows_b.at[ch % NBUF], sem_g)

    gd = [None] * NCHUNK
    sd = [None] * NCHUNK
    for i in range(min(NBUF - 2, NCHUNK)):
        gd[i] = gstart(i)
    for ch in range(NCHUNK):
        if ch >= 2:
            sd[ch - 2].wait()
        nxt = ch + NBUF - 2
        if nxt < NCHUNK:
            gd[nxt] = gstart(nxt)
        gd[ch].wait()
        sd[ch] = pltpu.async_copy(
            rows_b.at[ch % NBUF], acc_sh.at[seg2d.at[ch]], sem_s, add=True
        )
    for ch in range(max(NCHUNK - 2, 0), NCHUNK):
        sd[ch].wait()

    plsc.subcore_barrier()

    pltpu.sync_copy(acc_sh.at[pl.ds(slot * SLOT + h * L, L)], cmb_v)
    for ll in range(L // 2):
        for j in range(D // 16):
            out_v[ll, pl.ds(j * 16, 16)] = (
                cmb_v[2 * ll, pl.ds(j * 16, 16)]
                + cmb_v[2 * ll + 1, pl.ds(D + j * 16, 16)]
            )
    pltpu.sync_copy(out_v, out.at[b, pl.ds(h * (L // 2), L // 2)])


@jax.jit
def _run(para, slen, wt):
    w2 = _build_w2(wt)
    mesh = plsc.VectorSubcoreMesh(
        core_axis_name="c", subcore_axis_name="s", num_cores=NC, num_subcores=NS
    )
    zeros = jnp.zeros((SLOT, 2 * D), jnp.float32)
    f = pl.kernel(
        _body,
        out_type=jax.ShapeDtypeStruct((B, L, D), jnp.float32),
        mesh=mesh,
        compiler_params=pltpu.CompilerParams(needs_layout_passes=False),
        scratch_types=[
            pltpu.VMEM((L,), jnp.int32),
            pltpu.VMEM((L,), jnp.int32),
            pltpu.VMEM((HALF,), jnp.int32),
            pltpu.VMEM((NCHUNK, CH), jnp.int32),
            pltpu.VMEM((HALF,), jnp.int32),
            pltpu.VMEM((HALF,), jnp.int32),
            pltpu.VMEM((NBUF, CH, 2 * D), jnp.float32),
            pltpu.VMEM((L, 2 * D), jnp.float32),
            pltpu.VMEM((L // 2, D), jnp.float32),
            pltpu.VMEM_SHARED((RPC * SLOT, 2 * D), jnp.float32),
            pltpu.SemaphoreType.DMA,
            pltpu.SemaphoreType.DMA,
        ],
    )
    return f(para, slen, zeros, w2)


def kernel(paragraph_variable, sentence_length_list, max_no_lines, W):
    del max_no_lines
    para = paragraph_variable.astype(jnp.int32)
    slen = sentence_length_list.astype(jnp.int32)
    return _run(para, slen, W.T)


# speedup vs baseline: 3.0071x; 1.2250x over previous
import functools

import jax
import jax.numpy as jnp
from jax import lax
from jax.experimental import pallas as pl
from jax.experimental.pallas import tpu as pltpu
from jax.experimental.pallas import tpu_sc as plsc

B = 16
T = 4096
D = 64
L = 128
V = 1000000
NC = 2
NS = 16
RPC = B // NC
HALF = T // 2
CH = 128
NCHUNK = HALF // CH
NBUF = 4
SLOT = 2 * (L + 1) + 6
VMAIN = (V // 4096) * 4096
UMAIN = VMAIN // 2
HT = (V - VMAIN) // 2


def _mbody(x_ref, y_ref):
    for j in range(2):
        x = x_ref[:, pl.ds(j * 4096, 4096)]
        y_ref[pl.ds(j * 2048, 2048), :] = jnp.concatenate(
            [jnp.transpose(x[:, 0:2048]), jnp.transpose(x[:, 2048:4096])],
            axis=1)


def _tailbody(w2_ref, xt_ref, y_ref):
    del w2_ref
    xt = jnp.transpose(xt_ref[...])
    y_ref[0:HT, :] = jnp.concatenate([xt[0:HT], xt[HT:2 * HT]], axis=1)


def _build_w2(wt):
    w2a = pl.pallas_call(
        _mbody,
        grid=(VMAIN // 8192,),
        in_specs=[
            pl.BlockSpec((D, 8192), lambda i: (0, i)),
        ],
        out_specs=pl.BlockSpec((4096, 2 * D), lambda i: (i, 0)),
        out_shape=jax.ShapeDtypeStruct((V // 2, 2 * D), jnp.float32),
        compiler_params=pltpu.CompilerParams(
            dimension_semantics=("arbitrary",)),
    )(wt)
    return pl.pallas_call(
        _tailbody,
        grid=(1,),
        in_specs=[
            pl.BlockSpec(memory_space=pltpu.HBM),
            pl.BlockSpec((D, 1024), lambda i: (0, VMAIN // 1024)),
        ],
        out_specs=pl.BlockSpec((512, 2 * D), lambda i: (UMAIN // 512, 0)),
        out_shape=jax.ShapeDtypeStruct((V // 2, 2 * D), jnp.float32),
        input_output_aliases={0: 0},
        compiler_params=pltpu.CompilerParams(
            dimension_semantics=("arbitrary",)),
    )(w2a, wt)


def _body(para, slen, zeros, w2, out, len_v, bnd_v, mark_v, seg2d, tok_v, tok2_v,
          rows_b, cmb_v, out_v, acc_sh, sem_g, sem_s):
    c = lax.axis_index("c")
    s = lax.axis_index("s")
    slot = lax.rem(s, RPC)
    h = s // RPC
    b = c * RPC + slot
    t0 = h * HALF

    @pl.when(h == 0)
    def _():
        pltpu.sync_copy(zeros, acc_sh.at[pl.ds(slot * SLOT, SLOT)])

    pltpu.sync_copy(slen.at[b], len_v)
    pltpu.sync_copy(para.at[b, pl.ds(t0, HALF)], tok_v)

    carry = jnp.int32(0)
    for k in range(L // 16):
        v = len_v[pl.ds(k * 16, 16)]
        bnd_v[pl.ds(k * 16, 16)] = plsc.cumsum(v) + carry
        carry = carry + jnp.sum(v)

    off = jnp.int32(0)
    for k in range(L // 16):
        bv = bnd_v[pl.ds(k * 16, 16)]
        off = off + jnp.sum((bv <= t0 - 1).astype(jnp.int32))

    zero16 = jnp.zeros((16,), jnp.int32)
    for k in range(HALF // 16):
        mark_v[pl.ds(k * 16, 16)] = zero16

    lane = lax.iota(jnp.int32, 16)
    shift_idx = jnp.minimum(lane + 1, 15)
    for k in range(L // 16):
        v = bnd_v[pl.ds(k * 16, 16)]
        nxt = v.at[shift_idx].get(mode="promise_in_bounds")
        if k < L // 16 - 1:
            nv = bnd_v[pl.ds((k + 1) * 16, 16)]
            nxt = jnp.where(lane == 15, nv[0], nxt)
        else:
            nxt = jnp.where(lane == 15, jnp.int32(0x7FFFFFFF), nxt)
        rel = v - t0
        m = (v != nxt) & (rel >= 0) & (rel < HALF)
        relc = jnp.clip(rel, 0, HALF - 1)
        plsc.store_scatter(mark_v, [relc], lane + (16 * k + 1), mask=m)

    carry2 = off
    base = slot * SLOT
    for k in range(HALF // 16):
        v = mark_v[pl.ds(k * 16, 16)]
        cm = jnp.maximum(plsc.cummax(v), carry2)
        tv = tok_v[pl.ds(k * 16, 16)]
        vm = tv < VMAIN
        um = ((tv >> 12) << 11) + jnp.bitwise_and(tv, 2047)
        r = tv - VMAIN
        pt = (r >= HT).astype(jnp.int32)
        ut = UMAIN + r - HT * pt
        tok2_v[pl.ds(k * 16, 16)] = jnp.where(vm, um, ut)
        p = jnp.where(vm, jnp.bitwise_and(tv >> 11, 1), pt)
        seg2d[k // 8, pl.ds((k % 8) * 16, 16)] = cm * 2 + base + p
        carry2 = jnp.max(cm)

    plsc.subcore_barrier()

    def gstart(ch):
        idx = tok2_v.at[pl.ds(ch * CH, CH)]
        return pltpu.async_copy(w2.at[idx], rows_b.at[ch % NBUF], sem_g)

    gd = [None] * NCHUNK
    sd = [None] * NCHUNK
    for i in range(min(NBUF - 2, NCHUNK)):
        gd[i] = gstart(i)
    for ch in range(NCHUNK):
        if ch >= 2:
            sd[ch - 2].wait()
        nxt = ch + NBUF - 2
        if nxt < NCHUNK:
            gd[nxt] = gstart(nxt)
        gd[ch].wait()
        sd[ch] = pltpu.async_copy(
            rows_b.at[ch % NBUF], acc_sh.at[seg2d.at[ch]], sem_s, add=True
        )
    for ch in range(max(NCHUNK - 2, 0), NCHUNK):
        sd[ch].wait()

    plsc.subcore_barrier()

    pltpu.sync_copy(acc_sh.at[pl.ds(slot * SLOT + h * L, L)], cmb_v)
    for ll in range(L // 2):
        for j in range(D // 16):
            out_v[ll, pl.ds(j * 16, 16)] = (
                cmb_v[2 * ll, pl.ds(j * 16, 16)]
                + cmb_v[2 * ll + 1, pl.ds(D + j * 16, 16)]
            )
    pltpu.sync_copy(out_v, out.at[b, pl.ds(h * (L // 2), L // 2)])


@jax.jit
def _run(para, slen, wt):
    w2 = _build_w2(wt)
    mesh = plsc.VectorSubcoreMesh(
        core_axis_name="c", subcore_axis_name="s", num_cores=NC, num_subcores=NS
    )
    zeros = jnp.zeros((SLOT, 2 * D), jnp.float32)
    f = pl.kernel(
        _body,
        out_type=jax.ShapeDtypeStruct((B, L, D), jnp.float32),
        mesh=mesh,
        compiler_params=pltpu.CompilerParams(needs_layout_passes=False),
        scratch_types=[
            pltpu.VMEM((L,), jnp.int32),
            pltpu.VMEM((L,), jnp.int32),
            pltpu.VMEM((HALF,), jnp.int32),
            pltpu.VMEM((NCHUNK, CH), jnp.int32),
            pltpu.VMEM((HALF,), jnp.int32),
            pltpu.VMEM((HALF,), jnp.int32),
            pltpu.VMEM((NBUF, CH, 2 * D), jnp.float32),
            pltpu.VMEM((L, 2 * D), jnp.float32),
            pltpu.VMEM((L // 2, D), jnp.float32),
            pltpu.VMEM_SHARED((RPC * SLOT, 2 * D), jnp.float32),
            pltpu.SemaphoreType.DMA,
            pltpu.SemaphoreType.DMA,
        ],
    )
    return f(para, slen, zeros, w2)


def kernel(paragraph_variable, sentence_length_list, max_no_lines, W):
    del max_no_lines
    para = paragraph_variable.astype(jnp.int32)
    slen = sentence_length_list.astype(jnp.int32)
    return _run(para, slen, W.T)


# speedup vs baseline: 3.3861x; 1.1261x over previous
import functools

import jax
import jax.numpy as jnp
from jax import lax
from jax.experimental import pallas as pl
from jax.experimental.pallas import tpu as pltpu
from jax.experimental.pallas import tpu_sc as plsc

B = 16
T = 4096
D = 64
L = 128
V = 1000000
NC = 2
NS = 16
RPC = B // NC
HALF = T // 2
CH = 128
NCHUNK = HALF // CH
NBUF = 4
SLOT = 2 * (L + 1) + 6
VMAIN = (V // 4096) * 4096
UMAIN = VMAIN // 2
HT = (V - VMAIN) // 2


def _mbody(x_ref, y_ref):
    for j in range(4):
        x = x_ref[:, pl.ds(j * 4096, 4096)]
        y_ref[pl.ds(j * 2048, 2048), :] = jnp.concatenate(
            [jnp.transpose(x[:, 0:2048]), jnp.transpose(x[:, 2048:4096])],
            axis=1)


def _tailbody(w2_ref, xt_ref, y_ref):
    del w2_ref
    xt = jnp.transpose(xt_ref[...])
    y_ref[0:HT, :] = jnp.concatenate([xt[0:HT], xt[HT:2 * HT]], axis=1)


def _build_w2(wt):
    w2a = pl.pallas_call(
        _mbody,
        grid=(VMAIN // 16384,),
        in_specs=[
            pl.BlockSpec((D, 16384), lambda i: (0, i)),
        ],
        out_specs=pl.BlockSpec((8192, 2 * D), lambda i: (i, 0)),
        out_shape=jax.ShapeDtypeStruct((V // 2, 2 * D), jnp.float32),
        compiler_params=pltpu.CompilerParams(
            dimension_semantics=("arbitrary",)),
    )(wt)
    return pl.pallas_call(
        _tailbody,
        grid=(1,),
        in_specs=[
            pl.BlockSpec(memory_space=pltpu.HBM),
            pl.BlockSpec((D, 1024), lambda i: (0, VMAIN // 1024)),
        ],
        out_specs=pl.BlockSpec((512, 2 * D), lambda i: (UMAIN // 512, 0)),
        out_shape=jax.ShapeDtypeStruct((V // 2, 2 * D), jnp.float32),
        input_output_aliases={0: 0},
        compiler_params=pltpu.CompilerParams(
            dimension_semantics=("arbitrary",)),
    )(w2a, wt)


def _body(para, slen, zeros, w2, out, len_v, bnd_v, mark_v, seg2d, tok_v, tok2_v,
          rows_b, cmb_v, out_v, acc_sh, sem_g, sem_s):
    c = lax.axis_index("c")
    s = lax.axis_index("s")
    slot = lax.rem(s, RPC)
    h = s // RPC
    b = c * RPC + slot
    t0 = h * HALF

    @pl.when(h == 0)
    def _():
        pltpu.sync_copy(zeros, acc_sh.at[pl.ds(slot * SLOT, SLOT)])

    pltpu.sync_copy(slen.at[b], len_v)
    pltpu.sync_copy(para.at[b, pl.ds(t0, HALF)], tok_v)

    carry = jnp.int32(0)
    for k in range(L // 16):
        v = len_v[pl.ds(k * 16, 16)]
        bnd_v[pl.ds(k * 16, 16)] = plsc.cumsum(v) + carry
        carry = carry + jnp.sum(v)

    off = jnp.int32(0)
    for k in range(L // 16):
        bv = bnd_v[pl.ds(k * 16, 16)]
        off = off + jnp.sum((bv <= t0 - 1).astype(jnp.int32))

    zero16 = jnp.zeros((16,), jnp.int32)
    for k in range(HALF // 16):
        mark_v[pl.ds(k * 16, 16)] = zero16

    lane = lax.iota(jnp.int32, 16)
    shift_idx = jnp.minimum(lane + 1, 15)
    for k in range(L // 16):
        v = bnd_v[pl.ds(k * 16, 16)]
        nxt = v.at[shift_idx].get(mode="promise_in_bounds")
        if k < L // 16 - 1:
            nv = bnd_v[pl.ds((k + 1) * 16, 16)]
            nxt = jnp.where(lane == 15, nv[0], nxt)
        else:
            nxt = jnp.where(lane == 15, jnp.int32(0x7FFFFFFF), nxt)
        rel = v - t0
        m = (v != nxt) & (rel >= 0) & (rel < HALF)
        relc = jnp.clip(rel, 0, HALF - 1)
        plsc.store_scatter(mark_v, [relc], lane + (16 * k + 1), mask=m)

    carry2 = off
    base = slot * SLOT
    for k in range(HALF // 16):
        v = mark_v[pl.ds(k * 16, 16)]
        cm = jnp.maximum(plsc.cummax(v), carry2)
        tv = tok_v[pl.ds(k * 16, 16)]
        vm = tv < VMAIN
        um = ((tv >> 12) << 11) + jnp.bitwise_and(tv, 2047)
        r = tv - VMAIN
        pt = (r >= HT).astype(jnp.int32)
        ut = UMAIN + r - HT * pt
        tok2_v[pl.ds(k * 16, 16)] = jnp.where(vm, um, ut)
        p = jnp.where(vm, jnp.bitwise_and(tv >> 11, 1), pt)
        seg2d[k // 8, pl.ds((k % 8) * 16, 16)] = cm * 2 + base + p
        carry2 = jnp.max(cm)

    plsc.subcore_barrier()

    def gstart(ch):
        idx = tok2_v.at[pl.ds(ch * CH, CH)]
        return pltpu.async_copy(w2.at[idx], rows_b.at[ch % NBUF], sem_g)

    gd = [None] * NCHUNK
    sd = [None] * NCHUNK
    for i in range(min(NBUF - 2, NCHUNK)):
        gd[i] = gstart(i)
    for ch in range(NCHUNK):
        if ch >= 2:
            sd[ch - 2].wait()
        nxt = ch + NBUF - 2
        if nxt < NCHUNK:
            gd[nxt] = gstart(nxt)
        gd[ch].wait()
        sd[ch] = pltpu.async_copy(
            rows_b.at[ch % NBUF], acc_sh.at[seg2d.at[ch]], sem_s, add=True
        )
    for ch in range(max(NCHUNK - 2, 0), NCHUNK):
        sd[ch].wait()

    plsc.subcore_barrier()

    pltpu.sync_copy(acc_sh.at[pl.ds(slot * SLOT + h * L, L)], cmb_v)
    for ll in range(L // 2):
        for j in range(D // 16):
            out_v[ll, pl.ds(j * 16, 16)] = (
                cmb_v[2 * ll, pl.ds(j * 16, 16)]
                + cmb_v[2 * ll + 1, pl.ds(D + j * 16, 16)]
            )
    pltpu.sync_copy(out_v, out.at[b, pl.ds(h * (L // 2), L // 2)])


@jax.jit
def _run(para, slen, wt):
    w2 = _build_w2(wt)
    mesh = plsc.VectorSubcoreMesh(
        core_axis_name="c", subcore_axis_name="s", num_cores=NC, num_subcores=NS
    )
    zeros = jnp.zeros((SLOT, 2 * D), jnp.float32)
    f = pl.kernel(
        _body,
        out_type=jax.ShapeDtypeStruct((B, L, D), jnp.float32),
        mesh=mesh,
        compiler_params=pltpu.CompilerParams(needs_layout_passes=False),
        scratch_types=[
            pltpu.VMEM((L,), jnp.int32),
            pltpu.VMEM((L,), jnp.int32),
            pltpu.VMEM((HALF,), jnp.int32),
            pltpu.VMEM((NCHUNK, CH), jnp.int32),
            pltpu.VMEM((HALF,), jnp.int32),
            pltpu.VMEM((HALF,), jnp.int32),
            pltpu.VMEM((NBUF, CH, 2 * D), jnp.float32),
            pltpu.VMEM((L, 2 * D), jnp.float32),
            pltpu.VMEM((L // 2, D), jnp.float32),
            pltpu.VMEM_SHARED((RPC * SLOT, 2 * D), jnp.float32),
            pltpu.SemaphoreType.DMA,
            pltpu.SemaphoreType.DMA,
        ],
    )
    return f(para, slen, zeros, w2)


def kernel(paragraph_variable, sentence_length_list, max_no_lines, W):
    del max_no_lines
    para = paragraph_variable.astype(jnp.int32)
    slen = sentence_length_list.astype(jnp.int32)
    return _run(para, slen, W.T)
